# baseline (speedup 1.0000x reference)
import functools

import jax
import jax.numpy as jnp
from jax.experimental import pallas as pl
from jax.experimental.pallas import tpu as pltpu


def _fwd_kernel(x_ref, maskt_ref, w1_ref, b1_ref,
                wt2_ref, b2_ref, wt3_ref, b3_ref, wt4_ref, b4_ref,
                wL_ref, bL_ref, wo_ref, bo_ref,
                o_ref, a_sc, b_sc, z_sc,
                *, seq, H, W, Hd, nb):
    Wp = W + 2
    Npad = a_sc.shape[1]
    P = H * Wp - 2
    off0 = Wp + 1
    F = a_sc.shape[2]
    taps = tuple(dy * Wp + dx for dy in range(3) for dx in range(3))

    maskt = maskt_ref[...]

    a_sc[...] = jnp.zeros_like(a_sc)
    b_sc[...] = jnp.zeros_like(b_sc)
    z_sc[...] = jnp.zeros_like(z_sc)

    def act_pieces(e):
        ps = []
        for o in taps:
            if o % 2 == 0:
                ps.append(a_sc[e, o:o + P, :])
            else:
                ps.append(b_sc[e, o - 1:o - 1 + P, :])
        return jnp.concatenate(ps, axis=1)

    cs = [jnp.zeros((Npad, Hd), jnp.float32) for _ in range(nb)]

    for t in range(seq):
        for e in range(nb):
            r1 = jnp.dot(x_ref[e, t], w1_ref[...],
                         preferred_element_type=jnp.float32)
            v = (jnp.maximum(r1 + b1_ref[...], 0.0).astype(jnp.bfloat16)
                 * maskt)
            a_sc[e, off0:off0 + P, :] = v
            b_sc[e, off0 - 1:off0 - 1 + P, :] = v

        for e in range(nb):
            for wt_ref, bb_ref, last in ((wt2_ref, b2_ref, False),
                                         (wt3_ref, b3_ref, False),
                                         (wt4_ref, b4_ref, True)):
                s = act_pieces(e)
                r = jnp.dot(s, wt_ref[...],
                            preferred_element_type=jnp.float32)
                v = (jnp.maximum(r + bb_ref[...], 0.0).astype(jnp.bfloat16)
                     * maskt)
                if last:
                    z_sc[e, off0:off0 + P, 0:F] = v
                else:
                    a_sc[e, off0:off0 + P, :] = v
                    b_sc[e, off0 - 1:off0 - 1 + P, :] = v

        for e in range(nb):
            zv = z_sc[e]

            def gate(k):
                return (jnp.dot(zv, wL_ref[:, k * Hd:(k + 1) * Hd],
                                preferred_element_type=jnp.float32)
                        + bL_ref[:, k * Hd:(k + 1) * Hd])
            i_g = jax.nn.sigmoid(gate(0))
            f_g = jax.nn.sigmoid(gate(1))
            g_g = jnp.tanh(gate(2))
            o_g = jax.nn.sigmoid(gate(3))
            cs[e] = f_g * cs[e] + i_g * g_g
            z_sc[e, :, F:2 * F] = (o_g * jnp.tanh(cs[e])).astype(jnp.bfloat16)

    for e in range(nb):
        h = z_sc[e, :, F:2 * F]
        o_ref[e] = (jax.lax.dot_general(wo_ref[...], h,
                                        (((1,), (1,)), ((), ())),
                                        preferred_element_type=jnp.float32)
                    + bo_ref[...])


def _fused_forward(xpad, maskt, w1, b1, wts, bs, wL, bL, wo, bo,
                   *, H, W, Hd, nb=2):
    b, seq = xpad.shape[:2]
    Npad = (H + 2) * (W + 2)
    F = w1.shape[1]
    Cout = wo.shape[0]

    in_specs = [
        pl.BlockSpec((nb,) + xpad.shape[1:], lambda n: (n, 0, 0, 0)),
        pl.BlockSpec(maskt.shape, lambda n: (0, 0)),
        pl.BlockSpec(w1.shape, lambda n: (0, 0)),
        pl.BlockSpec(b1.shape, lambda n: (0, 0)),
    ]
    inputs = [xpad, maskt, w1, b1]
    for wt, bv in zip(wts, bs):
        in_specs += [pl.BlockSpec(wt.shape, lambda n: (0, 0)),
                     pl.BlockSpec(bv.shape, lambda n: (0, 0))]
        inputs += [wt, bv]
    for arr in (wL, bL, wo, bo):
        in_specs.append(pl.BlockSpec(arr.shape, lambda n: (0, 0)))
        inputs.append(arr)

    kern = functools.partial(_fwd_kernel, seq=seq, H=H, W=W, Hd=Hd, nb=nb)
    return pl.pallas_call(
        kern,
        out_shape=jax.ShapeDtypeStruct((b, Cout, Npad), jnp.float32),
        grid_spec=pltpu.PrefetchScalarGridSpec(
            num_scalar_prefetch=0,
            grid=(b // nb,),
            in_specs=in_specs,
            out_specs=pl.BlockSpec((nb, Cout, Npad), lambda n: (n, 0, 0)),
            scratch_shapes=[pltpu.VMEM((nb, Npad, F), jnp.bfloat16),
                            pltpu.VMEM((nb, Npad, F), jnp.bfloat16),
                            pltpu.VMEM((nb, Npad, 2 * F), jnp.bfloat16)],
        ),
        compiler_params=pltpu.CompilerParams(
            dimension_semantics=("parallel",)),
    )(*inputs)


def _interior_mask_t(h, w, feats):
    Wp = w + 2
    P = h * Wp - 2
    q = jnp.arange(P, dtype=jnp.int32) + (Wp + 1)
    col = q % Wp
    row = q // Wp
    valid = (col >= 1) & (col <= w) & (row >= 1) & (row <= h)
    m = valid.astype(jnp.bfloat16).reshape(P, 1)
    return jnp.broadcast_to(m, (P, feats))


def kernel(enc1_w, enc1_b, enc2_w, enc2_b, enc3_w, enc3_b, enc4_w, enc4_b,
           wih, whh, b_lstm, wout, bout, peat_map, temporal_ft, static_ft):
    del peat_map
    b, t, seq, h, w = temporal_ft.shape
    k = static_ft.shape[1]
    cin = k + t

    t_ft = jnp.transpose(temporal_ft, (0, 2, 1, 3, 4))
    s_ft = jnp.broadcast_to(static_ft, (b, k, seq, h, w))
    s_ft = jnp.transpose(s_ft, (0, 2, 1, 3, 4))
    x = jnp.concatenate([s_ft, t_ft], axis=2).astype(jnp.float32)

    cpad = -(-cin // 16) * 16
    Wp, Npad, P = w + 2, (h + 2) * (w + 2), h * (w + 2) - 2
    taps = tuple(dy * Wp + dx for dy in range(3) for dx in range(3))
    xp = jnp.pad(x, ((0, 0), (0, 0), (0, cpad - cin), (1, 1), (1, 1)))
    xflat = xp.reshape(b, seq, cpad, Npad)
    sx = jnp.stack([xflat[:, :, :, o:o + P] for o in taps], axis=2)
    sx = jnp.transpose(sx, (0, 1, 4, 2, 3))
    sx = sx.reshape(b, seq, P, 9 * cpad).astype(jnp.bfloat16)

    feats = enc1_w.shape[-1]
    hd = whh.shape[0]

    w1f = jnp.pad(enc1_w, ((0, 0), (0, 0), (0, cpad - cin), (0, 0)))
    w1 = jnp.transpose(w1f, (3, 0, 1, 2)).reshape(feats, -1).T
    w1 = w1.astype(jnp.bfloat16)
    b1 = enc1_b.reshape(1, feats)

    wts, bs = [], []
    for wv, bv in ((enc2_w, enc2_b), (enc3_w, enc3_b), (enc4_w, enc4_b)):
        wt = jnp.transpose(wv, (3, 0, 1, 2)).reshape(feats, -1).T
        wts.append(wt.astype(jnp.bfloat16))
        bs.append(bv.reshape(1, feats))
    wL = jnp.concatenate([wih, whh], axis=0).astype(jnp.bfloat16)
    bL = b_lstm.reshape(1, 4 * hd)
    wo = wout.T.astype(jnp.bfloat16)
    bo = bout.reshape(-1, 1)

    out = _fused_forward(sx, _interior_mask_t(h, w, feats), w1, b1, wts, bs,
                         wL, bL, wo, bo, H=h, W=w, Hd=hd)
    out_ch = wo.shape[0]
    out = out.reshape(b, out_ch, h + 2, w + 2)[:, :, 1:h + 1, 1:w + 1]
    return out[:, :, None]


# speedup vs baseline: 1.1869x; 1.1869x over previous
import functools

import jax
import jax.numpy as jnp
from jax.experimental import pallas as pl
from jax.experimental.pallas import tpu as pltpu


def _fwd_kernel(x_ref, maskt_ref, w1_ref, b1_ref,
                wt2_ref, b2_ref, wt3_ref, b3_ref, wt4_ref, b4_ref,
                wL_ref, bL_ref, wo_ref, bo_ref,
                o_ref, a_sc, b_sc, z_sc, c1a_sc, c1b_sc,
                *, seq, H, W, Hd, nb):
    Wp = W + 2
    Npad = a_sc.shape[1]
    P = H * Wp - 2
    off0 = Wp + 1
    F = a_sc.shape[2]
    taps = tuple(dy * Wp + dx for dy in range(3) for dx in range(3))

    maskt = maskt_ref[...]

    a_sc[...] = jnp.zeros_like(a_sc)
    b_sc[...] = jnp.zeros_like(b_sc)
    z_sc[...] = jnp.zeros_like(z_sc)
    c1a_sc[...] = jnp.zeros_like(c1a_sc)
    c1b_sc[...] = jnp.zeros_like(c1b_sc)

    for e in range(nb):
        for t in range(seq):
            s1 = jnp.concatenate([x_ref[e, t, :, o:o + P] for o in taps],
                                 axis=0)
            r1 = jax.lax.dot_general(s1, w1_ref[...],
                                     (((0,), (0,)), ((), ())),
                                     preferred_element_type=jnp.float32)
            v = (jnp.maximum(r1 + b1_ref[...], 0.0).astype(jnp.bfloat16)
                 * maskt)
            c1a_sc[e, t, off0:off0 + P, :] = v
            c1b_sc[e, t, off0 - 1:off0 - 1 + P, :] = v

    def act_pieces(e, t, first):
        ps = []
        for o in taps:
            if o % 2 == 0:
                ps.append(c1a_sc[e, t, o:o + P, :] if first
                          else a_sc[e, o:o + P, :])
            else:
                ps.append(c1b_sc[e, t, o - 1:o - 1 + P, :] if first
                          else b_sc[e, o - 1:o - 1 + P, :])
        return jnp.concatenate(ps, axis=1)

    cs = [jnp.zeros((Npad, Hd), jnp.float32) for _ in range(nb)]

    for t in range(seq):
        for e in range(nb):
            for wt_ref, bb_ref, first, last in (
                    (wt2_ref, b2_ref, True, False),
                    (wt3_ref, b3_ref, False, False),
                    (wt4_ref, b4_ref, False, True)):
                s = act_pieces(e, t, first)
                r = jnp.dot(s, wt_ref[...],
                            preferred_element_type=jnp.float32)
                v = (jnp.maximum(r + bb_ref[...], 0.0).astype(jnp.bfloat16)
                     * maskt)
                if last:
                    z_sc[e, off0:off0 + P, 0:F] = v
                else:
                    a_sc[e, off0:off0 + P, :] = v
                    b_sc[e, off0 - 1:off0 - 1 + P, :] = v

        for e in range(nb):
            zv = z_sc[e]

            def gate(k):
                return (jnp.dot(zv, wL_ref[:, k * Hd:(k + 1) * Hd],
                                preferred_element_type=jnp.float32)
                        + bL_ref[:, k * Hd:(k + 1) * Hd])
            i_g = jax.nn.sigmoid(gate(0))
            f_g = jax.nn.sigmoid(gate(1))
            g_g = jnp.tanh(gate(2))
            o_g = jax.nn.sigmoid(gate(3))
            cs[e] = f_g * cs[e] + i_g * g_g
            z_sc[e, :, F:2 * F] = (o_g * jnp.tanh(cs[e])).astype(jnp.bfloat16)

    for e in range(nb):
        h = z_sc[e, :, F:2 * F]
        o_ref[e] = (jax.lax.dot_general(wo_ref[...], h,
                                        (((1,), (1,)), ((), ())),
                                        preferred_element_type=jnp.float32)
                    + bo_ref[...])


def _fused_forward(xpad, maskt, w1, b1, wts, bs, wL, bL, wo, bo,
                   *, H, W, Hd, nb=2):
    b, seq = xpad.shape[:2]
    Npad = (H + 2) * (W + 2)
    F = w1.shape[1]
    Cout = wo.shape[0]

    in_specs = [
        pl.BlockSpec((nb,) + xpad.shape[1:], lambda n: (n, 0, 0, 0)),
        pl.BlockSpec(maskt.shape, lambda n: (0, 0)),
        pl.BlockSpec(w1.shape, lambda n: (0, 0)),
        pl.BlockSpec(b1.shape, lambda n: (0, 0)),
    ]
    inputs = [xpad, maskt, w1, b1]
    for wt, bv in zip(wts, bs):
        in_specs += [pl.BlockSpec(wt.shape, lambda n: (0, 0)),
                     pl.BlockSpec(bv.shape, lambda n: (0, 0))]
        inputs += [wt, bv]
    for arr in (wL, bL, wo, bo):
        in_specs.append(pl.BlockSpec(arr.shape, lambda n: (0, 0)))
        inputs.append(arr)

    kern = functools.partial(_fwd_kernel, seq=seq, H=H, W=W, Hd=Hd, nb=nb)
    return pl.pallas_call(
        kern,
        out_shape=jax.ShapeDtypeStruct((b, Cout, Npad), jnp.float32),
        grid_spec=pltpu.PrefetchScalarGridSpec(
            num_scalar_prefetch=0,
            grid=(b // nb,),
            in_specs=in_specs,
            out_specs=pl.BlockSpec((nb, Cout, Npad), lambda n: (n, 0, 0)),
            scratch_shapes=[pltpu.VMEM((nb, Npad, F), jnp.bfloat16),
                            pltpu.VMEM((nb, Npad, F), jnp.bfloat16),
                            pltpu.VMEM((nb, Npad, 2 * F), jnp.bfloat16),
                            pltpu.VMEM((nb, seq, Npad, F), jnp.bfloat16),
                            pltpu.VMEM((nb, seq, Npad, F), jnp.bfloat16)],
        ),
        compiler_params=pltpu.CompilerParams(
            dimension_semantics=("parallel",)),
    )(*inputs)


def _interior_mask_t(h, w, feats):
    Wp = w + 2
    P = h * Wp - 2
    q = jnp.arange(P, dtype=jnp.int32) + (Wp + 1)
    col = q % Wp
    row = q // Wp
    valid = (col >= 1) & (col <= w) & (row >= 1) & (row <= h)
    m = valid.astype(jnp.bfloat16).reshape(P, 1)
    return jnp.broadcast_to(m, (P, feats))


def kernel(enc1_w, enc1_b, enc2_w, enc2_b, enc3_w, enc3_b, enc4_w, enc4_b,
           wih, whh, b_lstm, wout, bout, peat_map, temporal_ft, static_ft):
    del peat_map
    b, t, seq, h, w = temporal_ft.shape
    k = static_ft.shape[1]
    cin = k + t

    t_ft = jnp.transpose(temporal_ft, (0, 2, 1, 3, 4))
    s_ft = jnp.broadcast_to(static_ft, (b, k, seq, h, w))
    s_ft = jnp.transpose(s_ft, (0, 2, 1, 3, 4))
    x = jnp.concatenate([s_ft, t_ft], axis=2).astype(jnp.float32)

    cpad = -(-cin // 16) * 16
    xp = jnp.pad(x, ((0, 0), (0, 0), (0, cpad - cin), (1, 1), (1, 1)))
    xp = xp.reshape(b, seq, cpad, (h + 2) * (w + 2)).astype(jnp.bfloat16)

    feats = enc1_w.shape[-1]
    hd = whh.shape[0]

    w1f = jnp.pad(enc1_w, ((0, 0), (0, 0), (0, cpad - cin), (0, 0)))
    w1 = jnp.transpose(w1f, (3, 0, 1, 2)).reshape(feats, -1).T
    w1 = w1.astype(jnp.bfloat16)
    b1 = enc1_b.reshape(1, feats)

    wts, bs = [], []
    for wv, bv in ((enc2_w, enc2_b), (enc3_w, enc3_b), (enc4_w, enc4_b)):
        wt = jnp.transpose(wv, (3, 0, 1, 2)).reshape(feats, -1).T
        wts.append(wt.astype(jnp.bfloat16))
        bs.append(bv.reshape(1, feats))
    wL = jnp.concatenate([wih, whh], axis=0).astype(jnp.bfloat16)
    bL = b_lstm.reshape(1, 4 * hd)
    wo = wout.T.astype(jnp.bfloat16)
    bo = bout.reshape(-1, 1)

    out = _fused_forward(xp, _interior_mask_t(h, w, feats), w1, b1, wts, bs,
                         wL, bL, wo, bo, H=h, W=w, Hd=hd)
    out_ch = wo.shape[0]
    out = out.reshape(b, out_ch, h + 2, w + 2)[:, :, 1:h + 1, 1:w + 1]
    return out[:, :, None]


# speedup vs baseline: 1.2413x; 1.0458x over previous
import functools

import jax
import jax.numpy as jnp
from jax.experimental import pallas as pl
from jax.experimental.pallas import tpu as pltpu


def _fwd_kernel(x_ref, maskt_ref, w1_ref, b1_ref,
                wt2_ref, b2_ref, wt3_ref, b3_ref, wt4_ref, b4_ref,
                wL_ref, bL_ref, wo_ref, bo_ref,
                o_ref, a_sc, b_sc, z_sc, c1a_sc, c1b_sc,
                *, seq, H, W, Hd, nb):
    Wp = W + 2
    Npad = a_sc.shape[1]
    P = H * Wp - 2
    off0 = Wp + 1
    F = a_sc.shape[2]
    taps = tuple(dy * Wp + dx for dy in range(3) for dx in range(3))

    maskt = maskt_ref[...]

    a_sc[...] = jnp.zeros_like(a_sc)
    b_sc[...] = jnp.zeros_like(b_sc)
    z_sc[...] = jnp.zeros_like(z_sc)
    c1a_sc[...] = jnp.zeros_like(c1a_sc)
    c1b_sc[...] = jnp.zeros_like(c1b_sc)

    def conv1(e, t):
        s1 = jnp.concatenate([x_ref[e, t, :, o:o + P] for o in taps],
                             axis=0)
        r1 = jax.lax.dot_general(s1, w1_ref[...],
                                 (((0,), (0,)), ((), ())),
                                 preferred_element_type=jnp.float32)
        v = (jnp.maximum(r1 + b1_ref[...], 0.0).astype(jnp.bfloat16)
             * maskt)
        p = t % 2
        c1a_sc[e, p, off0:off0 + P, :] = v
        c1b_sc[e, p, off0 - 1:off0 - 1 + P, :] = v

    def act_pieces(e, t, first):
        ps = []
        for o in taps:
            if o % 2 == 0:
                ps.append(c1a_sc[e, t % 2, o:o + P, :] if first
                          else a_sc[e, o:o + P, :])
            else:
                ps.append(c1b_sc[e, t % 2, o - 1:o - 1 + P, :] if first
                          else b_sc[e, o - 1:o - 1 + P, :])
        return jnp.concatenate(ps, axis=1)

    cs = [jnp.zeros((Npad, Hd), jnp.float32) for _ in range(nb)]

    for e in range(nb):
        conv1(e, 0)
    for t in range(seq):
        for e in range(nb):
            if t + 1 < seq:
                conv1(e, t + 1)
        for e in range(nb):
            for wt_ref, bb_ref, first, last in (
                    (wt2_ref, b2_ref, True, False),
                    (wt3_ref, b3_ref, False, False),
                    (wt4_ref, b4_ref, False, True)):
                s = act_pieces(e, t, first)
                r = jnp.dot(s, wt_ref[...],
                            preferred_element_type=jnp.float32)
                v = (jnp.maximum(r + bb_ref[...], 0.0).astype(jnp.bfloat16)
                     * maskt)
                if last:
                    z_sc[e, off0:off0 + P, 0:F] = v
                else:
                    a_sc[e, off0:off0 + P, :] = v
                    b_sc[e, off0 - 1:off0 - 1 + P, :] = v

        for e in range(nb):
            zv = z_sc[e]

            def gate(k):
                return (jnp.dot(zv, wL_ref[:, k * Hd:(k + 1) * Hd],
                                preferred_element_type=jnp.float32)
                        + bL_ref[:, k * Hd:(k + 1) * Hd])
            i_g = jax.nn.sigmoid(gate(0))
            f_g = jax.nn.sigmoid(gate(1))
            g_g = jnp.tanh(gate(2))
            o_g = jax.nn.sigmoid(gate(3))
            cs[e] = f_g * cs[e] + i_g * g_g
            z_sc[e, :, F:2 * F] = (o_g * jnp.tanh(cs[e])).astype(jnp.bfloat16)

    for e in range(nb):
        h = z_sc[e, :, F:2 * F]
        o_ref[e] = (jax.lax.dot_general(wo_ref[...], h,
                                        (((1,), (1,)), ((), ())),
                                        preferred_element_type=jnp.float32)
                    + bo_ref[...])


def _fused_forward(xpad, maskt, w1, b1, wts, bs, wL, bL, wo, bo,
                   *, H, W, Hd, nb=2):
    b, seq = xpad.shape[:2]
    Npad = (H + 2) * (W + 2)
    F = w1.shape[1]
    Cout = wo.shape[0]

    in_specs = [
        pl.BlockSpec((nb,) + xpad.shape[1:], lambda n: (n, 0, 0, 0)),
        pl.BlockSpec(maskt.shape, lambda n: (0, 0)),
        pl.BlockSpec(w1.shape, lambda n: (0, 0)),
        pl.BlockSpec(b1.shape, lambda n: (0, 0)),
    ]
    inputs = [xpad, maskt, w1, b1]
    for wt, bv in zip(wts, bs):
        in_specs += [pl.BlockSpec(wt.shape, lambda n: (0, 0)),
                     pl.BlockSpec(bv.shape, lambda n: (0, 0))]
        inputs += [wt, bv]
    for arr in (wL, bL, wo, bo):
        in_specs.append(pl.BlockSpec(arr.shape, lambda n: (0, 0)))
        inputs.append(arr)

    kern = functools.partial(_fwd_kernel, seq=seq, H=H, W=W, Hd=Hd, nb=nb)
    return pl.pallas_call(
        kern,
        out_shape=jax.ShapeDtypeStruct((b, Cout, Npad), jnp.float32),
        grid_spec=pltpu.PrefetchScalarGridSpec(
            num_scalar_prefetch=0,
            grid=(b // nb,),
            in_specs=in_specs,
            out_specs=pl.BlockSpec((nb, Cout, Npad), lambda n: (n, 0, 0)),
            scratch_shapes=[pltpu.VMEM((nb, Npad, F), jnp.bfloat16),
                            pltpu.VMEM((nb, Npad, F), jnp.bfloat16),
                            pltpu.VMEM((nb, Npad, 2 * F), jnp.bfloat16),
                            pltpu.VMEM((nb, 2, Npad, F), jnp.bfloat16),
                            pltpu.VMEM((nb, 2, Npad, F), jnp.bfloat16)],
        ),
        compiler_params=pltpu.CompilerParams(
            dimension_semantics=("parallel",)),
    )(*inputs)


def _interior_mask_t(h, w, feats):
    Wp = w + 2
    P = h * Wp - 2
    q = jnp.arange(P, dtype=jnp.int32) + (Wp + 1)
    col = q % Wp
    row = q // Wp
    valid = (col >= 1) & (col <= w) & (row >= 1) & (row <= h)
    m = valid.astype(jnp.bfloat16).reshape(P, 1)
    return jnp.broadcast_to(m, (P, feats))


def kernel(enc1_w, enc1_b, enc2_w, enc2_b, enc3_w, enc3_b, enc4_w, enc4_b,
           wih, whh, b_lstm, wout, bout, peat_map, temporal_ft, static_ft):
    del peat_map
    b, t, seq, h, w = temporal_ft.shape
    k = static_ft.shape[1]
    cin = k + t

    t_ft = jnp.transpose(temporal_ft, (0, 2, 1, 3, 4))
    s_ft = jnp.broadcast_to(static_ft, (b, k, seq, h, w))
    s_ft = jnp.transpose(s_ft, (0, 2, 1, 3, 4))
    x = jnp.concatenate([s_ft, t_ft], axis=2).astype(jnp.float32)

    cpad = -(-cin // 16) * 16
    xp = jnp.pad(x, ((0, 0), (0, 0), (0, cpad - cin), (1, 1), (1, 1)))
    xp = xp.reshape(b, seq, cpad, (h + 2) * (w + 2)).astype(jnp.bfloat16)

    feats = enc1_w.shape[-1]
    hd = whh.shape[0]

    w1f = jnp.pad(enc1_w, ((0, 0), (0, 0), (0, cpad - cin), (0, 0)))
    w1 = jnp.transpose(w1f, (3, 0, 1, 2)).reshape(feats, -1).T
    w1 = w1.astype(jnp.bfloat16)
    b1 = enc1_b.reshape(1, feats)

    wts, bs = [], []
    for wv, bv in ((enc2_w, enc2_b), (enc3_w, enc3_b), (enc4_w, enc4_b)):
        wt = jnp.transpose(wv, (3, 0, 1, 2)).reshape(feats, -1).T
        wts.append(wt.astype(jnp.bfloat16))
        bs.append(bv.reshape(1, feats))
    wL = jnp.concatenate([wih, whh], axis=0).astype(jnp.bfloat16)
    bL = b_lstm.reshape(1, 4 * hd)
    wo = wout.T.astype(jnp.bfloat16)
    bo = bout.reshape(-1, 1)

    out = _fused_forward(xp, _interior_mask_t(h, w, feats), w1, b1, wts, bs,
                         wL, bL, wo, bo, H=h, W=w, Hd=hd)
    out_ch = wo.shape[0]
    out = out.reshape(b, out_ch, h + 2, w + 2)[:, :, 1:h + 1, 1:w + 1]
    return out[:, :, None]


# speedup vs baseline: 1.3451x; 1.0836x over previous
import functools

import jax
import jax.numpy as jnp
from jax.experimental import pallas as pl
from jax.experimental.pallas import tpu as pltpu


def _fwd_kernel(x_ref, maskt_ref, w1_ref, b1_ref,
                wt2_ref, b2_ref, wt3_ref, b3_ref, wt4_ref, b4_ref,
                wL_ref, bL_ref, wo_ref, bo_ref,
                o_ref, a_sc, b_sc, z_sc, c1a_sc, c1b_sc,
                *, seq, H, W, Hd, nb):
    Wp = W + 2
    Npad = a_sc.shape[1]
    P = H * Wp - 2
    off0 = Wp + 1
    F = a_sc.shape[2]
    taps = tuple(dy * Wp + dx for dy in range(3) for dx in range(3))

    maskt = maskt_ref[...]

    a_sc[...] = jnp.zeros_like(a_sc)
    b_sc[...] = jnp.zeros_like(b_sc)
    z_sc[...] = jnp.zeros_like(z_sc)
    c1a_sc[...] = jnp.zeros_like(c1a_sc)
    c1b_sc[...] = jnp.zeros_like(c1b_sc)

    def conv1(e, t):
        s1 = jnp.concatenate([x_ref[e, t, :, o:o + P] for o in taps],
                             axis=0)
        r1 = jax.lax.dot_general(s1, w1_ref[...],
                                 (((0,), (0,)), ((), ())),
                                 preferred_element_type=jnp.float32)
        v = (jnp.maximum(r1 + b1_ref[...], 0.0).astype(jnp.bfloat16)
             * maskt)
        p = t % 2
        c1a_sc[e, p, off0:off0 + P, :] = v
        c1b_sc[e, p, off0 - 1:off0 - 1 + P, :] = v

    def act_pieces(e, t, first):
        ps = []
        for o in taps:
            if o % 2 == 0:
                ps.append(c1a_sc[e, t % 2, o:o + P, :] if first
                          else a_sc[e, o:o + P, :])
            else:
                ps.append(c1b_sc[e, t % 2, o - 1:o - 1 + P, :] if first
                          else b_sc[e, o - 1:o - 1 + P, :])
        return jnp.concatenate(ps, axis=1)

    cs = [jnp.zeros((Npad, Hd), jnp.float32) for _ in range(nb)]

    for e in range(nb):
        conv1(e, 0)
    for t in range(seq):
        for e in range(nb):
            if t + 1 < seq:
                conv1(e, t + 1)
        for e in range(nb):
            for wt_ref, bb_ref, first, last in (
                    (wt2_ref, b2_ref, True, False),
                    (wt3_ref, b3_ref, False, False),
                    (wt4_ref, b4_ref, False, True)):
                s = act_pieces(e, t, first)
                r = jnp.dot(s, wt_ref[...],
                            preferred_element_type=jnp.float32)
                v = (jnp.maximum(r + bb_ref[...], 0.0).astype(jnp.bfloat16)
                     * maskt)
                if last:
                    z_sc[e, off0:off0 + P, 0:F] = v
                else:
                    a_sc[e, off0:off0 + P, :] = v
                    b_sc[e, off0 - 1:off0 - 1 + P, :] = v

        for e in range(nb):
            zv = z_sc[e]

            def gate(k):
                return (jnp.dot(zv, wL_ref[:, k * Hd:(k + 1) * Hd],
                                preferred_element_type=jnp.float32)
                        + bL_ref[:, k * Hd:(k + 1) * Hd])
            i_g = jax.nn.sigmoid(gate(0))
            f_g = jax.nn.sigmoid(gate(1))
            g_g = jnp.tanh(gate(2))
            o_g = jax.nn.sigmoid(gate(3))
            cs[e] = f_g * cs[e] + i_g * g_g
            z_sc[e, :, F:2 * F] = (o_g * jnp.tanh(cs[e])).astype(jnp.bfloat16)

    for e in range(nb):
        h = z_sc[e, :, F:2 * F]
        o_ref[e] = (jax.lax.dot_general(wo_ref[...], h,
                                        (((1,), (1,)), ((), ())),
                                        preferred_element_type=jnp.float32)
                    + bo_ref[...])


def _fused_forward(xpad, maskt, w1, b1, wts, bs, wL, bL, wo, bo,
                   *, H, W, Hd, nb=4):
    b, seq = xpad.shape[:2]
    Npad = (H + 2) * (W + 2)
    F = w1.shape[1]
    Cout = wo.shape[0]

    in_specs = [
        pl.BlockSpec((nb,) + xpad.shape[1:], lambda n: (n, 0, 0, 0)),
        pl.BlockSpec(maskt.shape, lambda n: (0, 0)),
        pl.BlockSpec(w1.shape, lambda n: (0, 0)),
        pl.BlockSpec(b1.shape, lambda n: (0, 0)),
    ]
    inputs = [xpad, maskt, w1, b1]
    for wt, bv in zip(wts, bs):
        in_specs += [pl.BlockSpec(wt.shape, lambda n: (0, 0)),
                     pl.BlockSpec(bv.shape, lambda n: (0, 0))]
        inputs += [wt, bv]
    for arr in (wL, bL, wo, bo):
        in_specs.append(pl.BlockSpec(arr.shape, lambda n: (0, 0)))
        inputs.append(arr)

    kern = functools.partial(_fwd_kernel, seq=seq, H=H, W=W, Hd=Hd, nb=nb)
    return pl.pallas_call(
        kern,
        out_shape=jax.ShapeDtypeStruct((b, Cout, Npad), jnp.float32),
        grid_spec=pltpu.PrefetchScalarGridSpec(
            num_scalar_prefetch=0,
            grid=(b // nb,),
            in_specs=in_specs,
            out_specs=pl.BlockSpec((nb, Cout, Npad), lambda n: (n, 0, 0)),
            scratch_shapes=[pltpu.VMEM((nb, Npad, F), jnp.bfloat16),
                            pltpu.VMEM((nb, Npad, F), jnp.bfloat16),
                            pltpu.VMEM((nb, Npad, 2 * F), jnp.bfloat16),
                            pltpu.VMEM((nb, 2, Npad, F), jnp.bfloat16),
                            pltpu.VMEM((nb, 2, Npad, F), jnp.bfloat16)],
        ),
        compiler_params=pltpu.CompilerParams(
            dimension_semantics=("parallel",)),
    )(*inputs)


def _interior_mask_t(h, w, feats):
    Wp = w + 2
    P = h * Wp - 2
    q = jnp.arange(P, dtype=jnp.int32) + (Wp + 1)
    col = q % Wp
    row = q // Wp
    valid = (col >= 1) & (col <= w) & (row >= 1) & (row <= h)
    m = valid.astype(jnp.bfloat16).reshape(P, 1)
    return jnp.broadcast_to(m, (P, feats))


def kernel(enc1_w, enc1_b, enc2_w, enc2_b, enc3_w, enc3_b, enc4_w, enc4_b,
           wih, whh, b_lstm, wout, bout, peat_map, temporal_ft, static_ft):
    del peat_map
    b, t, seq, h, w = temporal_ft.shape
    k = static_ft.shape[1]
    cin = k + t

    t_ft = jnp.transpose(temporal_ft, (0, 2, 1, 3, 4))
    s_ft = jnp.broadcast_to(static_ft, (b, k, seq, h, w))
    s_ft = jnp.transpose(s_ft, (0, 2, 1, 3, 4))
    x = jnp.concatenate([s_ft, t_ft], axis=2).astype(jnp.float32)

    cpad = -(-cin // 16) * 16
    xp = jnp.pad(x, ((0, 0), (0, 0), (0, cpad - cin), (1, 1), (1, 1)))
    xp = xp.reshape(b, seq, cpad, (h + 2) * (w + 2)).astype(jnp.bfloat16)

    feats = enc1_w.shape[-1]
    hd = whh.shape[0]

    w1f = jnp.pad(enc1_w, ((0, 0), (0, 0), (0, cpad - cin), (0, 0)))
    w1 = jnp.transpose(w1f, (3, 0, 1, 2)).reshape(feats, -1).T
    w1 = w1.astype(jnp.bfloat16)
    b1 = enc1_b.reshape(1, feats)

    wts, bs = [], []
    for wv, bv in ((enc2_w, enc2_b), (enc3_w, enc3_b), (enc4_w, enc4_b)):
        wt = jnp.transpose(wv, (3, 0, 1, 2)).reshape(feats, -1).T
        wts.append(wt.astype(jnp.bfloat16))
        bs.append(bv.reshape(1, feats))
    wL = jnp.concatenate([wih, whh], axis=0).astype(jnp.bfloat16)
    bL = b_lstm.reshape(1, 4 * hd)
    wo = wout.T.astype(jnp.bfloat16)
    bo = bout.reshape(-1, 1)

    out = _fused_forward(xp, _interior_mask_t(h, w, feats), w1, b1, wts, bs,
                         wL, bL, wo, bo, H=h, W=w, Hd=hd)
    out_ch = wo.shape[0]
    out = out.reshape(b, out_ch, h + 2, w + 2)[:, :, 1:h + 1, 1:w + 1]
    return out[:, :, None]


# speedup vs baseline: 1.3830x; 1.0282x over previous
import functools

import jax
import jax.numpy as jnp
from jax.experimental import pallas as pl
from jax.experimental.pallas import tpu as pltpu


def _fwd_kernel(x_ref, maskt_ref, w1_ref, b1_ref,
                wt2_ref, b2_ref, wt3_ref, b3_ref, wt4_ref, b4_ref,
                wL_ref, bL_ref, wo_ref, bo_ref,
                o_ref, a_sc, b_sc, z_sc, c1a_sc, c1b_sc,
                *, seq, H, W, Hd, nb):
    Wp = W + 2
    Npad = a_sc.shape[1]
    P = H * Wp - 2
    off0 = Wp + 1
    F = a_sc.shape[2]
    taps = tuple(dy * Wp + dx for dy in range(3) for dx in range(3))

    maskt = maskt_ref[...]

    a_sc[...] = jnp.zeros_like(a_sc)
    b_sc[...] = jnp.zeros_like(b_sc)
    z_sc[...] = jnp.zeros_like(z_sc)
    c1a_sc[...] = jnp.zeros_like(c1a_sc)
    c1b_sc[...] = jnp.zeros_like(c1b_sc)

    def conv1(e, t):
        s1 = jnp.concatenate([x_ref[e, t, :, o:o + P] for o in taps],
                             axis=0)
        r1 = jax.lax.dot_general(s1, w1_ref[...],
                                 (((0,), (0,)), ((), ())),
                                 preferred_element_type=jnp.float32)
        v = (jnp.maximum(r1 + b1_ref[...], 0.0).astype(jnp.bfloat16)
             * maskt)
        p = t % 2
        c1a_sc[e, p, off0:off0 + P, :] = v
        c1b_sc[e, p, off0 - 1:off0 - 1 + P, :] = v

    def act_pieces(e, t, first):
        ps = []
        for o in taps:
            if o % 2 == 0:
                ps.append(c1a_sc[e, t % 2, o:o + P, :] if first
                          else a_sc[e, o:o + P, :])
            else:
                ps.append(c1b_sc[e, t % 2, o - 1:o - 1 + P, :] if first
                          else b_sc[e, o - 1:o - 1 + P, :])
        return jnp.concatenate(ps, axis=1)

    cs = [jnp.zeros((Npad, Hd), jnp.float32) for _ in range(nb)]

    for e in range(nb):
        conv1(e, 0)
    for t in range(seq):
        for e in range(nb):
            if t + 1 < seq:
                conv1(e, t + 1)
        for e in range(nb):
            for wt_ref, bb_ref, first, last in (
                    (wt2_ref, b2_ref, True, False),
                    (wt3_ref, b3_ref, False, False),
                    (wt4_ref, b4_ref, False, True)):
                s = act_pieces(e, t, first)
                r = jnp.dot(s, wt_ref[...],
                            preferred_element_type=jnp.float32)
                v = (jnp.maximum(r + bb_ref[...], 0.0).astype(jnp.bfloat16)
                     * maskt)
                if last:
                    z_sc[e, off0:off0 + P, 0:F] = v
                else:
                    a_sc[e, off0:off0 + P, :] = v
                    b_sc[e, off0 - 1:off0 - 1 + P, :] = v

        for e in range(nb):
            zv = z_sc[e]

            def gate(k):
                return (jnp.dot(zv, wL_ref[:, k * Hd:(k + 1) * Hd],
                                preferred_element_type=jnp.float32)
                        + bL_ref[:, k * Hd:(k + 1) * Hd])
            i_g = jax.nn.sigmoid(gate(0))
            f_g = jax.nn.sigmoid(gate(1))
            g_g = jnp.tanh(gate(2))
            o_g = jax.nn.sigmoid(gate(3))
            cs[e] = f_g * cs[e] + i_g * g_g
            z_sc[e, :, F:2 * F] = (o_g * jnp.tanh(cs[e])).astype(jnp.bfloat16)

    for e in range(nb):
        h = z_sc[e, :, F:2 * F]
        o_ref[e] = (jax.lax.dot_general(wo_ref[...], h,
                                        (((1,), (1,)), ((), ())),
                                        preferred_element_type=jnp.float32)
                    + bo_ref[...])


def _fused_forward(xpad, maskt, w1, b1, wts, bs, wL, bL, wo, bo,
                   *, H, W, Hd, nb=8):
    b, seq = xpad.shape[:2]
    Npad = (H + 2) * (W + 2)
    F = w1.shape[1]
    Cout = wo.shape[0]

    in_specs = [
        pl.BlockSpec((nb,) + xpad.shape[1:], lambda n: (n, 0, 0, 0)),
        pl.BlockSpec(maskt.shape, lambda n: (0, 0)),
        pl.BlockSpec(w1.shape, lambda n: (0, 0)),
        pl.BlockSpec(b1.shape, lambda n: (0, 0)),
    ]
    inputs = [xpad, maskt, w1, b1]
    for wt, bv in zip(wts, bs):
        in_specs += [pl.BlockSpec(wt.shape, lambda n: (0, 0)),
                     pl.BlockSpec(bv.shape, lambda n: (0, 0))]
        inputs += [wt, bv]
    for arr in (wL, bL, wo, bo):
        in_specs.append(pl.BlockSpec(arr.shape, lambda n: (0, 0)))
        inputs.append(arr)

    kern = functools.partial(_fwd_kernel, seq=seq, H=H, W=W, Hd=Hd, nb=nb)
    return pl.pallas_call(
        kern,
        out_shape=jax.ShapeDtypeStruct((b, Cout, Npad), jnp.float32),
        grid_spec=pltpu.PrefetchScalarGridSpec(
            num_scalar_prefetch=0,
            grid=(b // nb,),
            in_specs=in_specs,
            out_specs=pl.BlockSpec((nb, Cout, Npad), lambda n: (n, 0, 0)),
            scratch_shapes=[pltpu.VMEM((nb, Npad, F), jnp.bfloat16),
                            pltpu.VMEM((nb, Npad, F), jnp.bfloat16),
                            pltpu.VMEM((nb, Npad, 2 * F), jnp.bfloat16),
                            pltpu.VMEM((nb, 2, Npad, F), jnp.bfloat16),
                            pltpu.VMEM((nb, 2, Npad, F), jnp.bfloat16)],
        ),
        compiler_params=pltpu.CompilerParams(
            dimension_semantics=("parallel",)),
    )(*inputs)


def _interior_mask_t(h, w, feats):
    Wp = w + 2
    P = h * Wp - 2
    q = jnp.arange(P, dtype=jnp.int32) + (Wp + 1)
    col = q % Wp
    row = q // Wp
    valid = (col >= 1) & (col <= w) & (row >= 1) & (row <= h)
    m = valid.astype(jnp.bfloat16).reshape(P, 1)
    return jnp.broadcast_to(m, (P, feats))


def kernel(enc1_w, enc1_b, enc2_w, enc2_b, enc3_w, enc3_b, enc4_w, enc4_b,
           wih, whh, b_lstm, wout, bout, peat_map, temporal_ft, static_ft):
    del peat_map
    b, t, seq, h, w = temporal_ft.shape
    k = static_ft.shape[1]
    cin = k + t

    t_ft = jnp.transpose(temporal_ft, (0, 2, 1, 3, 4))
    s_ft = jnp.broadcast_to(static_ft, (b, k, seq, h, w))
    s_ft = jnp.transpose(s_ft, (0, 2, 1, 3, 4))
    x = jnp.concatenate([s_ft, t_ft], axis=2).astype(jnp.float32)

    cpad = -(-cin // 16) * 16
    xp = jnp.pad(x, ((0, 0), (0, 0), (0, cpad - cin), (1, 1), (1, 1)))
    xp = xp.reshape(b, seq, cpad, (h + 2) * (w + 2)).astype(jnp.bfloat16)

    feats = enc1_w.shape[-1]
    hd = whh.shape[0]

    w1f = jnp.pad(enc1_w, ((0, 0), (0, 0), (0, cpad - cin), (0, 0)))
    w1 = jnp.transpose(w1f, (3, 0, 1, 2)).reshape(feats, -1).T
    w1 = w1.astype(jnp.bfloat16)
    b1 = enc1_b.reshape(1, feats)

    wts, bs = [], []
    for wv, bv in ((enc2_w, enc2_b), (enc3_w, enc3_b), (enc4_w, enc4_b)):
        wt = jnp.transpose(wv, (3, 0, 1, 2)).reshape(feats, -1).T
        wts.append(wt.astype(jnp.bfloat16))
        bs.append(bv.reshape(1, feats))
    wL = jnp.concatenate([wih, whh], axis=0).astype(jnp.bfloat16)
    bL = b_lstm.reshape(1, 4 * hd)
    wo = wout.T.astype(jnp.bfloat16)
    bo = bout.reshape(-1, 1)

    out = _fused_forward(xp, _interior_mask_t(h, w, feats), w1, b1, wts, bs,
                         wL, bL, wo, bo, H=h, W=w, Hd=hd)
    out_ch = wo.shape[0]
    out = out.reshape(b, out_ch, h + 2, w + 2)[:, :, 1:h + 1, 1:w + 1]
    return out[:, :, None]
